# Initial kernel scaffold; baseline (speedup 1.0000x reference)
import jax
import jax.numpy as jnp
from jax.experimental import pallas as pl


def kernel(x, edge_index, batch, W1, att_src, att_dst, b1, W2, b2, Wfc, bfc):
    raise NotImplementedError("write your pallas kernel here")


# speedup vs baseline: 18.8845x; 18.8845x over previous
import functools

import jax
import jax.numpy as jnp
from jax import lax
from jax.experimental import pallas as pl
from jax.experimental.pallas import tpu as pltpu
from jax.experimental.pallas import tpu_sc as plsc

N = 10000
E = 320000
D = 128
G = 64
NC = 2
NS = 16
NW = NC * NS
C = 128
CPT = -(-E // (NW * C))
EPT = CPT * C
EPAD = NW * EPT
NPAD = CPT * C

_f32 = jnp.float32
_i32 = jnp.int32


def _k1_body(x_ref, w1_ref, att2_ref, h1_ref, asd_ref):
    h = jnp.dot(x_ref[...], w1_ref[...], preferred_element_type=_f32)
    h1_ref[...] = h
    asd_ref[...] = jnp.dot(h, att2_ref[...], preferred_element_type=_f32)


_k1 = pl.pallas_call(
    _k1_body,
    out_shape=[
        jax.ShapeDtypeStruct((N, D), _f32),
        jax.ShapeDtypeStruct((N, 2), _f32),
    ],
)


def _k3_body(accA, accB, denA, denB, cntA, cntB, b1, w2, b2,
             h3s_ref, dinv_ref, sl_ref):
    deg = cntA[...] + cntB[...] + 1.0
    dinv = lax.rsqrt(deg)
    den = denA[...] + denB[...] + 1e-16
    gat = (accA[...] + accB[...]) / den + b1[...][None, :]
    h2 = jnp.maximum(gat, 0.0)
    h3 = jnp.dot(h2, w2[...], preferred_element_type=_f32)
    h3s = h3 * dinv
    h3s_ref[...] = h3s
    dinv_ref[...] = dinv
    sl_ref[...] = h3s * dinv + b2[...][None, :]


_k3 = pl.pallas_call(
    _k3_body,
    out_shape=[
        jax.ShapeDtypeStruct((N, D), _f32),
        jax.ShapeDtypeStruct((N, 1), _f32),
        jax.ShapeDtypeStruct((N, D), _f32),
    ],
)


_BLK = 2000
_NBLK = N // _BLK


def _k5_body(h4A, h4B, sl, batch_ref, wfc, bfc, out_ref, gsum, gcnt):
    i = pl.program_id(0)

    @pl.when(i == 0)
    def _():
        gsum[...] = jnp.zeros_like(gsum)
        gcnt[...] = jnp.zeros_like(gcnt)

    h4 = jnp.maximum(h4A[...] + h4B[...] + sl[...], 0.0)
    gids = lax.broadcasted_iota(_i32, (G, _BLK), 0)
    onehot = (batch_ref[0] == gids).astype(_f32)
    gsum[...] += jnp.dot(onehot, h4, preferred_element_type=_f32)
    gcnt[...] += jnp.sum(onehot, axis=1, keepdims=True)

    @pl.when(i == _NBLK - 1)
    def _():
        g = gsum[...] / jnp.maximum(gcnt[...], 1.0)
        out_ref[...] = jnp.dot(g, wfc[...],
                               preferred_element_type=_f32) + bfc[...][None, :]


_k5 = pl.pallas_call(
    _k5_body,
    grid=(_NBLK,),
    in_specs=[
        pl.BlockSpec((_BLK, D), lambda i: (i, 0)),
        pl.BlockSpec((_BLK, D), lambda i: (i, 0)),
        pl.BlockSpec((_BLK, D), lambda i: (i, 0)),
        pl.BlockSpec((1, 1, _BLK), lambda i: (i, 0, 0)),
        pl.BlockSpec((D, 2), lambda i: (0, 0)),
        pl.BlockSpec((2,), lambda i: (0,)),
    ],
    out_specs=pl.BlockSpec((G, 2), lambda i: (0, 0)),
    out_shape=jax.ShapeDtypeStruct((G, 2), _f32),
    scratch_shapes=[pltpu.VMEM((G, D), _f32), pltpu.VMEM((G, 1), _f32)],
)


_mesh = plsc.VectorSubcoreMesh(core_axis_name="c", subcore_axis_name="s",
                               num_cores=NC, num_subcores=NS)


def _zero_rows(zrows_v):
    def body(r, _):
        for v in range(D // 16):
            zrows_v[r, pl.ds(v * 16, 16)] = jnp.zeros((16,), _f32)
        return 0
    lax.fori_loop(0, C, body, 0)


def _scale_rows(rows_v, w_v):
    def body(g, _):
        w16 = w_v[pl.ds(g * 16, 16)]
        for k in range(16):
            ws = w16[k]
            e = g * 16 + k
            for v in range(D // 16):
                s2 = pl.ds(v * 16, 16)
                rows_v[e, s2] = rows_v[e, s2] * ws
        return 0
    lax.fori_loop(0, C // 16, body, 0)


@functools.partial(
    pl.kernel,
    out_type=[
        jax.ShapeDtypeStruct((NC * NPAD, D), _f32),
        jax.ShapeDtypeStruct((NC * NPAD,), _f32),
        jax.ShapeDtypeStruct((NC * NPAD,), _f32),
    ],
    mesh=_mesh,
    scratch_types=[
        pltpu.VMEM_SHARED((NPAD, D), _f32),
        pltpu.VMEM_SHARED((NPAD,), _f32),
        pltpu.VMEM_SHARED((NPAD,), _f32),
        pltpu.VMEM((CPT, C), _i32),
        pltpu.VMEM((CPT, C), _i32),
        pltpu.VMEM((C,), _f32),
        pltpu.VMEM((C,), _f32),
        pltpu.VMEM((C,), _f32),
        pltpu.VMEM((C,), _f32),
        pltpu.VMEM((C, D), _f32),
        pltpu.SemaphoreType.DMA,
        pltpu.SemaphoreType.DMA,
        pltpu.SemaphoreType.DMA,
    ],
)
def _gat_edge(h1, asrc, adst, srcp, dstp,
              acc_out, den_out, cnt_out,
              s_acc, s_den, s_cnt, src_v, dst_v, a_v, b_v, w_v, c_v,
              rows_v, sem0, sem1, sem2):
    cid = lax.axis_index("c")
    sid = lax.axis_index("s")
    wid = cid * NS + sid

    _zero_rows(rows_v)
    for j in range(CPT):
        @pl.when((j % NS) == sid)
        def _(jj=j):
            pltpu.sync_copy(rows_v, s_acc.at[pl.ds(jj * C, C)])
            pltpu.sync_copy(rows_v.at[0], s_den.at[pl.ds(jj * C, C)])
            pltpu.sync_copy(rows_v.at[0], s_cnt.at[pl.ds(jj * C, C)])
    pltpu.sync_copy(srcp.at[wid], src_v)
    pltpu.sync_copy(dstp.at[wid], dst_v)
    plsc.subcore_barrier()

    base_e = wid * EPT

    def chunk(j, _):
        d1 = pltpu.async_copy(h1.at[src_v.at[j]], rows_v, sem0)
        d2 = pltpu.async_copy(asrc.at[src_v.at[j]], a_v, sem1)
        d3 = pltpu.async_copy(adst.at[dst_v.at[j]], b_v, sem2)
        d2.wait()
        d3.wait()
        base = base_e + j * C
        for v in range(C // 16):
            sl = pl.ds(v * 16, 16)
            e16 = a_v[sl] + b_v[sl]
            e16 = jnp.where(e16 >= 0.0, e16, e16 * 0.2)
            w16 = jnp.exp(e16)
            gid = base + v * 16 + lax.iota(_i32, 16)
            valid = gid < E
            w_v[sl] = jnp.where(valid, w16, 0.0)
            c_v[sl] = jnp.where(valid, 1.0, 0.0)
        d1.wait()
        _scale_rows(rows_v, w_v)
        pltpu.sync_copy(rows_v, s_acc.at[dst_v.at[j]], add=True)
        pltpu.sync_copy(w_v, s_den.at[dst_v.at[j]], add=True)
        pltpu.sync_copy(c_v, s_cnt.at[dst_v.at[j]], add=True)
        return 0

    lax.fori_loop(0, CPT, chunk, 0)
    plsc.subcore_barrier()

    for j in range(CPT):
        @pl.when((j % NS) == sid)
        def _(jj=j):
            off = cid * NPAD + jj * C
            pltpu.sync_copy(s_acc.at[pl.ds(jj * C, C)],
                            acc_out.at[pl.ds(off, C)])
            pltpu.sync_copy(s_den.at[pl.ds(jj * C, C)],
                            den_out.at[pl.ds(off, C)])
            pltpu.sync_copy(s_cnt.at[pl.ds(jj * C, C)],
                            cnt_out.at[pl.ds(off, C)])


@functools.partial(
    pl.kernel,
    out_type=jax.ShapeDtypeStruct((NC * NPAD, D), _f32),
    mesh=_mesh,
    scratch_types=[
        pltpu.VMEM_SHARED((NPAD, D), _f32),
        pltpu.VMEM((CPT, C), _i32),
        pltpu.VMEM((CPT, C), _i32),
        pltpu.VMEM((C,), _f32),
        pltpu.VMEM((C,), _f32),
        pltpu.VMEM((C, D), _f32),
        pltpu.SemaphoreType.DMA,
        pltpu.SemaphoreType.DMA,
    ],
)
def _gcn_edge(h3s, dinv, srcp, dstp, acc_out,
              s_acc, src_v, dst_v, b_v, w_v, rows_v, sem0, sem1):
    cid = lax.axis_index("c")
    sid = lax.axis_index("s")
    wid = cid * NS + sid

    _zero_rows(rows_v)
    for j in range(CPT):
        @pl.when((j % NS) == sid)
        def _(jj=j):
            pltpu.sync_copy(rows_v, s_acc.at[pl.ds(jj * C, C)])
    pltpu.sync_copy(srcp.at[wid], src_v)
    pltpu.sync_copy(dstp.at[wid], dst_v)
    plsc.subcore_barrier()

    base_e = wid * EPT

    def chunk(j, _):
        d1 = pltpu.async_copy(h3s.at[src_v.at[j]], rows_v, sem0)
        d2 = pltpu.async_copy(dinv.at[dst_v.at[j]], b_v, sem1)
        d2.wait()
        base = base_e + j * C
        for v in range(C // 16):
            sl = pl.ds(v * 16, 16)
            gid = base + v * 16 + lax.iota(_i32, 16)
            valid = gid < E
            w_v[sl] = jnp.where(valid, b_v[sl], 0.0)
        d1.wait()
        _scale_rows(rows_v, w_v)
        pltpu.sync_copy(rows_v, s_acc.at[dst_v.at[j]], add=True)
        return 0

    lax.fori_loop(0, CPT, chunk, 0)
    plsc.subcore_barrier()

    for j in range(CPT):
        @pl.when((j % NS) == sid)
        def _(jj=j):
            off = cid * NPAD + jj * C
            pltpu.sync_copy(s_acc.at[pl.ds(jj * C, C)],
                            acc_out.at[pl.ds(off, C)])


def kernel(x, edge_index, batch, W1, att_src, att_dst, b1, W2, b2, Wfc, bfc):
    att2 = jnp.stack([att_src, att_dst], axis=1)
    h1, asd = _k1(x, W1, att2)
    a_src = asd[:, 0]
    a_dst = asd[:, 1]

    pad = jnp.zeros((EPAD - E,), _i32)
    srcp = jnp.concatenate([edge_index[0], pad]).reshape(NW, CPT, C)
    dstp = jnp.concatenate([edge_index[1], pad]).reshape(NW, CPT, C)

    acc, den, cnt = _gat_edge(h1, a_src, a_dst, srcp, dstp)
    h3s, dinv, sl = _k3(
        acc[:N], acc[NPAD:NPAD + N],
        den[:N, None], den[NPAD:NPAD + N, None],
        cnt[:N, None], cnt[NPAD:NPAD + N, None],
        b1, W2, b2,
    )

    acc2 = _gcn_edge(h3s, dinv[:, 0], srcp, dstp)
    batch3 = batch.reshape(_NBLK, 1, _BLK)
    return _k5(acc2[:N], acc2[NPAD:NPAD + N], sl, batch3, Wfc, bfc)
